# baseline (speedup 1.0000x reference)
import functools

import jax
import jax.numpy as jnp
from jax import lax
from jax.experimental import pallas as pl
from jax.experimental.pallas import tpu as pltpu
from jax.experimental.pallas import tpu_sc as plsc

_NC = 2
_NS = 16
_NW = _NC * _NS
_BLK = 128


def _sc_mesh():
    return plsc.VectorSubcoreMesh(core_axis_name="c", subcore_axis_name="s")


_SC_PARAMS = pltpu.CompilerParams(use_tc_tiling_on_sc=False,
                                  needs_layout_passes=False)


def _make_sc_degree(nblk, N2):
    nb = nblk // _NW
    nx = nblk - nb * _NW
    sl = N2 // _NS

    @functools.partial(
        pl.kernel,
        out_type=jax.ShapeDtypeStruct((_NC * N2,), jnp.float32),
        mesh=_sc_mesh(),
        compiler_params=_SC_PARAMS,
        scratch_types=[
            pltpu.VMEM((nb, _BLK), jnp.int32),
            pltpu.VMEM((1, _BLK), jnp.int32),
            pltpu.VMEM((_BLK,), jnp.float32),
            pltpu.VMEM_SHARED((N2,), jnp.float32),
            pltpu.SemaphoreType.DMA,
            pltpu.SemaphoreType.DMA,
        ],
    )
    def k(e3_hbm, zeros_hbm, ones_hbm, out_hbm, ridx, ridx_x, ones_v,
          deg_sh, semA, semB):
        c = lax.axis_index("c")
        s = lax.axis_index("s")
        w = c * _NS + s
        pltpu.sync_copy(zeros_hbm, deg_sh.at[pl.ds(s * sl, sl)])
        pltpu.sync_copy(ones_hbm, ones_v)
        pltpu.sync_copy(e3_hbm.at[0, pl.ds(w * nb, nb)], ridx)

        @pl.when(w < nx)
        def _():
            pltpu.sync_copy(e3_hbm.at[0, pl.ds(_NW * nb + w, 1)], ridx_x)

        plsc.subcore_barrier()

        def pair(p, carry):
            b0 = 2 * p
            cA = pltpu.async_copy(ones_v, deg_sh.at[ridx.at[b0]], semA,
                                  add=True)
            cB = pltpu.async_copy(ones_v, deg_sh.at[ridx.at[b0 + 1]], semB,
                                  add=True)
            cA.wait()
            cB.wait()
            return carry

        lax.fori_loop(0, nb // 2, pair, 0)
        if nb % 2:
            pltpu.async_copy(ones_v, deg_sh.at[ridx.at[nb - 1]], semA,
                             add=True).wait()

        @pl.when(w < nx)
        def _():
            pltpu.async_copy(ones_v, deg_sh.at[ridx_x.at[0]], semA,
                             add=True).wait()

        plsc.subcore_barrier()
        pltpu.sync_copy(deg_sh.at[pl.ds(s * sl, sl)],
                        out_hbm.at[pl.ds(c * N2 + s * sl, sl)])

    return k


def _make_sc_agg64(nblk, N2, H):
    nb = nblk // _NW
    nx = nblk - nb * _NW
    sl = N2 // _NS

    @functools.partial(
        pl.kernel,
        out_type=jax.ShapeDtypeStruct((_NC * N2, H), jnp.bfloat16),
        mesh=_sc_mesh(),
        compiler_params=_SC_PARAMS,
        scratch_types=[
            pltpu.VMEM((nb, _BLK), jnp.int32),
            pltpu.VMEM((nb, _BLK), jnp.int32),
            pltpu.VMEM((1, _BLK), jnp.int32),
            pltpu.VMEM((1, _BLK), jnp.int32),
            pltpu.VMEM((_BLK, H), jnp.bfloat16),
            pltpu.VMEM((_BLK, H), jnp.bfloat16),
            pltpu.VMEM((_BLK, H), jnp.bfloat16),
            pltpu.VMEM_SHARED((N2, H), jnp.bfloat16),
            pltpu.SemaphoreType.DMA,
            pltpu.SemaphoreType.DMA,
            pltpu.SemaphoreType.DMA,
        ],
    )
    def k(e3_hbm, tab_hbm, zeros_hbm, out_hbm,
          ridx, cidx, ridx_x, cidx_x, msgA, msgB, msgC, acc_sh,
          semA, semB, semC):
        c = lax.axis_index("c")
        s = lax.axis_index("s")
        w = c * _NS + s
        pltpu.sync_copy(zeros_hbm, acc_sh.at[pl.ds(s * sl, sl)])
        pltpu.sync_copy(e3_hbm.at[0, pl.ds(w * nb, nb)], ridx)
        pltpu.sync_copy(e3_hbm.at[1, pl.ds(w * nb, nb)], cidx)

        @pl.when(w < nx)
        def _():
            pltpu.sync_copy(e3_hbm.at[0, pl.ds(_NW * nb + w, 1)], ridx_x)
            pltpu.sync_copy(e3_hbm.at[1, pl.ds(_NW * nb + w, 1)], cidx_x)

        plsc.subcore_barrier()

        pltpu.async_copy(tab_hbm.at[cidx.at[0]], msgA, semA)
        pltpu.async_copy(tab_hbm.at[cidx.at[1]], msgB, semB)

        def tri(p, carry):
            b0 = 3 * p
            bufs = ((msgA, semA), (msgB, semB), (msgC, semC))
            for l in range(3):
                buf, sem = bufs[l]
                nxt = bufs[(l + 2) % 3]

                @pl.when(b0 + l + 2 < nb)
                def _(buf2=nxt[0], sem2=nxt[1], b=b0 + l + 2):
                    pltpu.async_copy(tab_hbm.at[cidx.at[b]], buf2, sem2)

                pltpu.make_async_copy(tab_hbm.at[cidx.at[b0 + l]], buf,
                                      sem).wait()
                pltpu.sync_copy(buf, acc_sh.at[ridx.at[b0 + l]], add=True)
            return carry

        lax.fori_loop(0, nb // 3, tri, 0)
        for l in range(nb - (nb // 3) * 3):
            b = (nb // 3) * 3 + l
            buf, sem = ((msgA, semA), (msgB, semB), (msgC, semC))[b % 3]
            pltpu.make_async_copy(tab_hbm.at[cidx.at[b]], buf, sem).wait()
            pltpu.sync_copy(buf, acc_sh.at[ridx.at[b]], add=True)

        @pl.when(w < nx)
        def _():
            pltpu.async_copy(tab_hbm.at[cidx_x.at[0]], msgA, semA).wait()
            pltpu.sync_copy(msgA, acc_sh.at[ridx_x.at[0]], add=True)

        plsc.subcore_barrier()
        pltpu.sync_copy(acc_sh.at[pl.ds(s * sl, sl)],
                        out_hbm.at[pl.ds(c * N2 + s * sl, sl)])

    return k


def _make_sc_agg1(nblk, N2):
    nb = nblk // _NW
    nx = nblk - nb * _NW
    sl = N2 // _NS

    @functools.partial(
        pl.kernel,
        out_type=jax.ShapeDtypeStruct((_NC * N2,), jnp.float32),
        mesh=_sc_mesh(),
        compiler_params=_SC_PARAMS,
        scratch_types=[
            pltpu.VMEM((nb, _BLK), jnp.int32),
            pltpu.VMEM((nb, _BLK), jnp.int32),
            pltpu.VMEM((1, _BLK), jnp.int32),
            pltpu.VMEM((1, _BLK), jnp.int32),
            pltpu.VMEM((N2,), jnp.float32),
            pltpu.VMEM((_BLK,), jnp.float32),
            pltpu.VMEM((_BLK,), jnp.float32),
            pltpu.VMEM_SHARED((N2,), jnp.float32),
            pltpu.SemaphoreType.DMA,
            pltpu.SemaphoreType.DMA,
        ],
    )
    def k(e3_hbm, tab_hbm, zeros_hbm, out_hbm,
          ridx, cidx, ridx_x, cidx_x, tab_v, valsA, valsB, acc_sh,
          semA, semB):
        c = lax.axis_index("c")
        s = lax.axis_index("s")
        w = c * _NS + s
        pltpu.sync_copy(zeros_hbm, acc_sh.at[pl.ds(s * sl, sl)])
        pltpu.sync_copy(tab_hbm, tab_v)
        pltpu.sync_copy(e3_hbm.at[0, pl.ds(w * nb, nb)], ridx)
        pltpu.sync_copy(e3_hbm.at[1, pl.ds(w * nb, nb)], cidx)

        @pl.when(w < nx)
        def _():
            pltpu.sync_copy(e3_hbm.at[0, pl.ds(_NW * nb + w, 1)], ridx_x)
            pltpu.sync_copy(e3_hbm.at[1, pl.ds(_NW * nb + w, 1)], cidx_x)

        plsc.subcore_barrier()

        def fill(idx_ref, b, dst):
            for i in range(_BLK // 16):
                iv = idx_ref[b, pl.ds(16 * i, 16)]
                dst[pl.ds(16 * i, 16)] = plsc.load_gather(tab_v, [iv])

        def pair(p, carry):
            b0 = 2 * p
            fill(cidx, b0, valsA)
            cA = pltpu.async_copy(valsA, acc_sh.at[ridx.at[b0]], semA,
                                  add=True)
            fill(cidx, b0 + 1, valsB)
            cB = pltpu.async_copy(valsB, acc_sh.at[ridx.at[b0 + 1]], semB,
                                  add=True)
            cA.wait()
            cB.wait()
            return carry

        lax.fori_loop(0, nb // 2, pair, 0)
        if nb % 2:
            fill(cidx, nb - 1, valsA)
            pltpu.async_copy(valsA, acc_sh.at[ridx.at[nb - 1]], semA,
                             add=True).wait()

        @pl.when(w < nx)
        def _():
            fill(cidx_x, 0, valsA)
            pltpu.async_copy(valsA, acc_sh.at[ridx_x.at[0]], semA,
                             add=True).wait()

        plsc.subcore_barrier()
        pltpu.sync_copy(acc_sh.at[pl.ds(s * sl, sl)],
                        out_hbm.at[pl.ds(c * N2 + s * sl, sl)])

    return k


def _tc_support(xp, W1, deg_p, N2, D, H, blk):
    g = N2 // blk

    def body(x_ref, w_ref, d0_ref, d1_ref, sup2_ref):
        sup = jnp.dot(x_ref[...], w_ref[...], preferred_element_type=jnp.float32)
        deg = 1.0 + d0_ref[...] + d1_ref[...]
        r = lax.rsqrt(deg)
        sup2_ref[...] = (sup * r[:, None]).astype(jnp.bfloat16)

    return pl.pallas_call(
        body,
        grid=(g,),
        in_specs=[
            pl.BlockSpec((blk, D), lambda i: (i, 0)),
            pl.BlockSpec((D, H), lambda i: (0, 0)),
            pl.BlockSpec((blk,), lambda i: (i,)),
            pl.BlockSpec((blk,), lambda i: (i + g,)),
        ],
        out_specs=pl.BlockSpec((blk, H), lambda i: (i, 0)),
        out_shape=jax.ShapeDtypeStruct((N2, H), jnp.bfloat16),
    )(xp, W1, deg_p, deg_p)


def _tc_mid(acc_p, sup2, deg_p, W2, Wd, N2, H, H2):
    def body(a0_ref, a1_ref, sup2_ref, d0_ref, d1_ref, w2_ref, wd_ref,
             s2_ref):
        deg = 1.0 + d0_ref[...] + d1_ref[...]
        r = lax.rsqrt(deg)
        pre = (a0_ref[...].astype(jnp.float32) + a1_ref[...].astype(jnp.float32)
               + sup2_ref[...].astype(jnp.float32))
        h1 = jnp.maximum(pre * r[:, None], 0.0)
        v = jnp.sum(w2_ref[...] * wd_ref[...][None, :], axis=1)
        s2_ref[...] = r * jnp.sum(h1 * v[None, :], axis=1)

    return pl.pallas_call(
        body,
        grid=(1,),
        in_specs=[
            pl.BlockSpec((N2, H), lambda i: (0, 0)),
            pl.BlockSpec((N2, H), lambda i: (1, 0)),
            pl.BlockSpec((N2, H), lambda i: (0, 0)),
            pl.BlockSpec((N2,), lambda i: (0,)),
            pl.BlockSpec((N2,), lambda i: (1,)),
            pl.BlockSpec((H, H2), lambda i: (0, 0)),
            pl.BlockSpec((H2,), lambda i: (0,)),
        ],
        out_specs=pl.BlockSpec((N2,), lambda i: (0,)),
        out_shape=jax.ShapeDtypeStruct((N2,), jnp.float32),
    )(acc_p, acc_p, sup2, deg_p, deg_p, W2, Wd)


def _tc_final(t_p, s2, deg_p, N, N2):
    def body(t0_ref, t1_ref, s2_ref, d0_ref, d1_ref, out_ref):
        deg = 1.0 + d0_ref[...] + d1_ref[...]
        r = lax.rsqrt(deg)
        z = (t0_ref[...] + t1_ref[...] + s2_ref[...]) * r
        out_ref[...] = jax.nn.sigmoid(z)[:N]

    return pl.pallas_call(
        body,
        grid=(1,),
        in_specs=[
            pl.BlockSpec((N2,), lambda i: (0,)),
            pl.BlockSpec((N2,), lambda i: (1,)),
            pl.BlockSpec((N2,), lambda i: (0,)),
            pl.BlockSpec((N2,), lambda i: (0,)),
            pl.BlockSpec((N2,), lambda i: (1,)),
        ],
        out_specs=pl.BlockSpec((N,), lambda i: (0,)),
        out_shape=jax.ShapeDtypeStruct((N,), jnp.float32),
    )(t_p, t_p, s2, deg_p, deg_p)


def kernel(x, edge_index, W1, W2, W3, Wd):
    del W3
    N, D = x.shape
    H = W1.shape[1]
    H2 = W2.shape[1]
    E = edge_index.shape[1]

    N2 = -(-N // 2048) * 2048
    blk = 2048

    assert E % _BLK == 0
    nblk = E // _BLK
    e3 = edge_index.reshape(2, nblk, _BLK)
    xp = jnp.zeros((N2, D), jnp.float32).at[:N].set(x)

    sl = N2 // _NS
    zeros1 = jnp.zeros((sl,), jnp.float32)
    zerosH = jnp.zeros((sl, H), jnp.bfloat16)
    ones = jnp.ones((_BLK,), jnp.float32)

    deg_p = _make_sc_degree(nblk, N2)(e3, zeros1, ones)

    sup2 = _tc_support(xp, W1, deg_p, N2, D, H, blk)

    acc_p = _make_sc_agg64(nblk, N2, H)(e3, sup2, zerosH)

    s2 = _tc_mid(acc_p, sup2, deg_p, W2, Wd, N2, H, H2)

    t_p = _make_sc_agg1(nblk, N2)(e3, s2, zeros1)

    return _tc_final(t_p, s2, deg_p, N, N2)


# speedup vs baseline: 1.0048x; 1.0048x over previous
import functools

import jax
import jax.numpy as jnp
from jax import lax
from jax.experimental import pallas as pl
from jax.experimental.pallas import tpu as pltpu
from jax.experimental.pallas import tpu_sc as plsc

_NC = 2
_NS = 16
_NW = _NC * _NS
_BLK = 128


def _sc_mesh():
    return plsc.VectorSubcoreMesh(core_axis_name="c", subcore_axis_name="s")


_SC_PARAMS = pltpu.CompilerParams(use_tc_tiling_on_sc=False,
                                  needs_layout_passes=False)


def _make_sc_degree(nblk, N2):
    nb = nblk // _NW
    nx = nblk - nb * _NW
    sl = N2 // _NS

    @functools.partial(
        pl.kernel,
        out_type=jax.ShapeDtypeStruct((_NC * N2,), jnp.float32),
        mesh=_sc_mesh(),
        compiler_params=_SC_PARAMS,
        scratch_types=[
            pltpu.VMEM((nb, _BLK), jnp.int32),
            pltpu.VMEM((1, _BLK), jnp.int32),
            pltpu.VMEM((_BLK,), jnp.float32),
            pltpu.VMEM_SHARED((N2,), jnp.float32),
            pltpu.SemaphoreType.DMA,
            pltpu.SemaphoreType.DMA,
        ],
    )
    def k(e3_hbm, zeros_hbm, ones_hbm, out_hbm, ridx, ridx_x, ones_v,
          deg_sh, semA, semB):
        c = lax.axis_index("c")
        s = lax.axis_index("s")
        w = c * _NS + s
        pltpu.sync_copy(zeros_hbm, deg_sh.at[pl.ds(s * sl, sl)])
        pltpu.sync_copy(ones_hbm, ones_v)
        pltpu.sync_copy(e3_hbm.at[0, pl.ds(w * nb, nb)], ridx)

        @pl.when(w < nx)
        def _():
            pltpu.sync_copy(e3_hbm.at[0, pl.ds(_NW * nb + w, 1)], ridx_x)

        plsc.subcore_barrier()

        def pair(p, carry):
            b0 = 2 * p
            cA = pltpu.async_copy(ones_v, deg_sh.at[ridx.at[b0]], semA,
                                  add=True)
            cB = pltpu.async_copy(ones_v, deg_sh.at[ridx.at[b0 + 1]], semB,
                                  add=True)
            cA.wait()
            cB.wait()
            return carry

        lax.fori_loop(0, nb // 2, pair, 0)
        if nb % 2:
            pltpu.async_copy(ones_v, deg_sh.at[ridx.at[nb - 1]], semA,
                             add=True).wait()

        @pl.when(w < nx)
        def _():
            pltpu.async_copy(ones_v, deg_sh.at[ridx_x.at[0]], semA,
                             add=True).wait()

        plsc.subcore_barrier()
        pltpu.sync_copy(deg_sh.at[pl.ds(s * sl, sl)],
                        out_hbm.at[pl.ds(c * N2 + s * sl, sl)])

    return k


def _make_sc_agg64(nblk, N2, H):
    nb = nblk // _NW
    nx = nblk - nb * _NW
    sl = N2 // _NS

    @functools.partial(
        pl.kernel,
        out_type=jax.ShapeDtypeStruct((_NC * N2, H), jnp.bfloat16),
        mesh=_sc_mesh(),
        compiler_params=_SC_PARAMS,
        scratch_types=[
            pltpu.VMEM((nb, _BLK), jnp.int32),
            pltpu.VMEM((nb, _BLK), jnp.int32),
            pltpu.VMEM((1, _BLK), jnp.int32),
            pltpu.VMEM((1, _BLK), jnp.int32),
            pltpu.VMEM((_BLK, H), jnp.bfloat16),
            pltpu.VMEM((_BLK, H), jnp.bfloat16),
            pltpu.VMEM((_BLK, H), jnp.bfloat16),
            pltpu.VMEM_SHARED((N2, H), jnp.bfloat16),
            pltpu.SemaphoreType.DMA,
            pltpu.SemaphoreType.DMA,
            pltpu.SemaphoreType.DMA,
        ],
    )
    def k(e3_hbm, tab_hbm, zeros_hbm, out_hbm,
          ridx, cidx, ridx_x, cidx_x, msgA, msgB, msgC, acc_sh,
          semA, semB, semC):
        c = lax.axis_index("c")
        s = lax.axis_index("s")
        w = c * _NS + s
        pltpu.sync_copy(zeros_hbm, acc_sh.at[pl.ds(s * sl, sl)])
        pltpu.sync_copy(e3_hbm.at[0, pl.ds(w * nb, nb)], ridx)
        pltpu.sync_copy(e3_hbm.at[1, pl.ds(w * nb, nb)], cidx)

        @pl.when(w < nx)
        def _():
            pltpu.sync_copy(e3_hbm.at[0, pl.ds(_NW * nb + w, 1)], ridx_x)
            pltpu.sync_copy(e3_hbm.at[1, pl.ds(_NW * nb + w, 1)], cidx_x)

        plsc.subcore_barrier()

        pltpu.async_copy(tab_hbm.at[cidx.at[0]], msgA, semA)
        pltpu.async_copy(tab_hbm.at[cidx.at[1]], msgB, semB)

        def tri(p, carry):
            b0 = 3 * p
            bufs = ((msgA, semA), (msgB, semB), (msgC, semC))
            for l in range(3):
                buf, sem = bufs[l]
                nxt = bufs[(l + 2) % 3]

                @pl.when(b0 + l + 2 < nb)
                def _(buf2=nxt[0], sem2=nxt[1], b=b0 + l + 2):
                    pltpu.async_copy(tab_hbm.at[cidx.at[b]], buf2, sem2)

                pltpu.make_async_copy(tab_hbm.at[cidx.at[b0 + l]], buf,
                                      sem).wait()
                pltpu.sync_copy(buf, acc_sh.at[ridx.at[b0 + l]], add=True)
            return carry

        lax.fori_loop(0, nb // 3, tri, 0)
        for l in range(nb - (nb // 3) * 3):
            b = (nb // 3) * 3 + l
            buf, sem = ((msgA, semA), (msgB, semB), (msgC, semC))[b % 3]
            pltpu.make_async_copy(tab_hbm.at[cidx.at[b]], buf, sem).wait()
            pltpu.sync_copy(buf, acc_sh.at[ridx.at[b]], add=True)

        @pl.when(w < nx)
        def _():
            pltpu.async_copy(tab_hbm.at[cidx_x.at[0]], msgA, semA).wait()
            pltpu.sync_copy(msgA, acc_sh.at[ridx_x.at[0]], add=True)

        plsc.subcore_barrier()
        pltpu.sync_copy(acc_sh.at[pl.ds(s * sl, sl)],
                        out_hbm.at[pl.ds(c * N2 + s * sl, sl)])

    return k


def _make_sc_agg1(nblk, N2):
    nb = nblk // _NW
    nx = nblk - nb * _NW
    sl = N2 // _NS

    @functools.partial(
        pl.kernel,
        out_type=jax.ShapeDtypeStruct((_NC * N2,), jnp.float32),
        mesh=_sc_mesh(),
        compiler_params=_SC_PARAMS,
        scratch_types=[
            pltpu.VMEM((nb, _BLK), jnp.int32),
            pltpu.VMEM((nb, _BLK), jnp.int32),
            pltpu.VMEM((1, _BLK), jnp.int32),
            pltpu.VMEM((1, _BLK), jnp.int32),
            pltpu.VMEM((N2,), jnp.float32),
            pltpu.VMEM((_BLK,), jnp.float32),
            pltpu.VMEM((_BLK,), jnp.float32),
            pltpu.VMEM_SHARED((N2,), jnp.float32),
            pltpu.SemaphoreType.DMA,
            pltpu.SemaphoreType.DMA,
        ],
    )
    def k(e3_hbm, tab_hbm, zeros_hbm, out_hbm,
          ridx, cidx, ridx_x, cidx_x, tab_v, valsA, valsB, acc_sh,
          semA, semB):
        c = lax.axis_index("c")
        s = lax.axis_index("s")
        w = c * _NS + s
        pltpu.sync_copy(zeros_hbm, acc_sh.at[pl.ds(s * sl, sl)])
        pltpu.sync_copy(tab_hbm, tab_v)
        pltpu.sync_copy(e3_hbm.at[0, pl.ds(w * nb, nb)], ridx)
        pltpu.sync_copy(e3_hbm.at[1, pl.ds(w * nb, nb)], cidx)

        @pl.when(w < nx)
        def _():
            pltpu.sync_copy(e3_hbm.at[0, pl.ds(_NW * nb + w, 1)], ridx_x)
            pltpu.sync_copy(e3_hbm.at[1, pl.ds(_NW * nb + w, 1)], cidx_x)

        plsc.subcore_barrier()

        def fill(idx_ref, b, dst):
            for i in range(_BLK // 16):
                iv = idx_ref[b, pl.ds(16 * i, 16)]
                dst[pl.ds(16 * i, 16)] = plsc.load_gather(tab_v, [iv])

        def pair(p, carry):
            b0 = 2 * p
            fill(cidx, b0, valsA)
            cA = pltpu.async_copy(valsA, acc_sh.at[ridx.at[b0]], semA,
                                  add=True)
            fill(cidx, b0 + 1, valsB)
            cB = pltpu.async_copy(valsB, acc_sh.at[ridx.at[b0 + 1]], semB,
                                  add=True)
            cA.wait()
            cB.wait()
            return carry

        lax.fori_loop(0, nb // 2, pair, 0)
        if nb % 2:
            fill(cidx, nb - 1, valsA)
            pltpu.async_copy(valsA, acc_sh.at[ridx.at[nb - 1]], semA,
                             add=True).wait()

        @pl.when(w < nx)
        def _():
            fill(cidx_x, 0, valsA)
            pltpu.async_copy(valsA, acc_sh.at[ridx_x.at[0]], semA,
                             add=True).wait()

        plsc.subcore_barrier()
        pltpu.sync_copy(acc_sh.at[pl.ds(s * sl, sl)],
                        out_hbm.at[pl.ds(c * N2 + s * sl, sl)])

    return k


def _tc_support(xp, W1, deg_p, N2, D, H, blk):
    g = N2 // blk

    def body(x_ref, w_ref, d0_ref, d1_ref, sup2_ref):
        sup = jnp.dot(x_ref[...], w_ref[...], preferred_element_type=jnp.float32)
        deg = 1.0 + d0_ref[...] + d1_ref[...]
        r = lax.rsqrt(deg)
        sup2_ref[...] = (sup * r[:, None]).astype(jnp.bfloat16)

    return pl.pallas_call(
        body,
        grid=(g,),
        in_specs=[
            pl.BlockSpec((blk, D), lambda i: (i, 0)),
            pl.BlockSpec((D, H), lambda i: (0, 0)),
            pl.BlockSpec((blk,), lambda i: (i,)),
            pl.BlockSpec((blk,), lambda i: (i + g,)),
        ],
        out_specs=pl.BlockSpec((blk, H), lambda i: (i, 0)),
        out_shape=jax.ShapeDtypeStruct((N2, H), jnp.bfloat16),
    )(xp, W1, deg_p, deg_p)


def _tc_mid(acc_p, sup2, deg_p, W2, Wd, N2, H, H2, blk):
    g = N2 // blk

    def body(a0_ref, a1_ref, sup2_ref, d0_ref, d1_ref, w2_ref, wd_ref,
             s2_ref):
        deg = 1.0 + d0_ref[...] + d1_ref[...]
        r = lax.rsqrt(deg)
        pre = (a0_ref[...].astype(jnp.float32) + a1_ref[...].astype(jnp.float32)
               + sup2_ref[...].astype(jnp.float32))
        h1 = jnp.maximum(pre * r[:, None], 0.0)
        v = jnp.sum(w2_ref[...] * wd_ref[...][None, :], axis=1)
        s2_ref[...] = r * jnp.sum(h1 * v[None, :], axis=1)

    return pl.pallas_call(
        body,
        grid=(g,),
        in_specs=[
            pl.BlockSpec((blk, H), lambda i: (i, 0)),
            pl.BlockSpec((blk, H), lambda i: (i + g, 0)),
            pl.BlockSpec((blk, H), lambda i: (i, 0)),
            pl.BlockSpec((blk,), lambda i: (i,)),
            pl.BlockSpec((blk,), lambda i: (i + g,)),
            pl.BlockSpec((H, H2), lambda i: (0, 0)),
            pl.BlockSpec((H2,), lambda i: (0,)),
        ],
        out_specs=pl.BlockSpec((blk,), lambda i: (i,)),
        out_shape=jax.ShapeDtypeStruct((N2,), jnp.float32),
    )(acc_p, acc_p, sup2, deg_p, deg_p, W2, Wd)


def _tc_final(t_p, s2, deg_p, N, N2):
    def body(t0_ref, t1_ref, s2_ref, d0_ref, d1_ref, out_ref):
        deg = 1.0 + d0_ref[...] + d1_ref[...]
        r = lax.rsqrt(deg)
        z = (t0_ref[...] + t1_ref[...] + s2_ref[...]) * r
        out_ref[...] = jax.nn.sigmoid(z)[:N]

    return pl.pallas_call(
        body,
        grid=(1,),
        in_specs=[
            pl.BlockSpec((N2,), lambda i: (0,)),
            pl.BlockSpec((N2,), lambda i: (1,)),
            pl.BlockSpec((N2,), lambda i: (0,)),
            pl.BlockSpec((N2,), lambda i: (0,)),
            pl.BlockSpec((N2,), lambda i: (1,)),
        ],
        out_specs=pl.BlockSpec((N,), lambda i: (0,)),
        out_shape=jax.ShapeDtypeStruct((N,), jnp.float32),
    )(t_p, t_p, s2, deg_p, deg_p)


def kernel(x, edge_index, W1, W2, W3, Wd):
    del W3
    N, D = x.shape
    H = W1.shape[1]
    H2 = W2.shape[1]
    E = edge_index.shape[1]

    N2 = -(-N // 2048) * 2048
    blk = 2048

    assert E % _BLK == 0
    nblk = E // _BLK
    e3 = edge_index.reshape(2, nblk, _BLK)
    xp = jnp.zeros((N2, D), jnp.float32).at[:N].set(x)

    sl = N2 // _NS
    zeros1 = jnp.zeros((sl,), jnp.float32)
    zerosH = jnp.zeros((sl, H), jnp.bfloat16)
    ones = jnp.ones((_BLK,), jnp.float32)

    deg_p = _make_sc_degree(nblk, N2)(e3, zeros1, ones)

    sup2 = _tc_support(xp, W1, deg_p, N2, D, H, blk)

    acc_p = _make_sc_agg64(nblk, N2, H)(e3, sup2, zerosH)

    s2 = _tc_mid(acc_p, sup2, deg_p, W2, Wd, N2, H, H2, blk)

    t_p = _make_sc_agg1(nblk, N2)(e3, s2, zeros1)

    return _tc_final(t_p, s2, deg_p, N, N2)
